# Initial kernel scaffold; baseline (speedup 1.0000x reference)
import jax
import jax.numpy as jnp
from jax.experimental import pallas as pl


def kernel(x, edge_index, fc0_W, fc0_b, pool_p, gru_Wih, gru_Whh, gru_bih, gru_bhh, init_W, fc_W, fc_b):
    raise NotImplementedError("write your pallas kernel here")


# speedup vs baseline: 4.6052x; 4.6052x over previous
import functools

import jax
import jax.numpy as jnp
from jax import lax
from jax.experimental import pallas as pl
from jax.experimental.pallas import tpu as pltpu
from jax.experimental.pallas import tpu_sc as plsc

N = 3000
F = 512
H = 512
E = 96000
NPAD = 3072
EPAD = 98304
NW = 32
EPW = EPAD // NW
EB = 128
NB = EPW // EB
RPS = NPAD // 16
NQ = 4
WQ = H // NQ

_MESH = plsc.VectorSubcoreMesh(core_axis_name="c", subcore_axis_name="s")


def _tc1_body(x_ref, w_ref, b_ref, x0_ref):
    x0 = lax.dot_general(x_ref[...], w_ref[...], (((1,), (1,)), ((), ())),
                         preferred_element_type=jnp.float32)
    x0_ref[...] = jnp.maximum(x0 + b_ref[...], 0.0)


def _tc1(x, fc0_W, fc0_b2):
    return pl.pallas_call(
        _tc1_body,
        out_shape=jax.ShapeDtypeStruct((N, F), jnp.float32),
    )(x, fc0_W, fc0_b2)


def _tc2_body(xtr_ref, tv_ref, wih_ref, whh_ref, bih_ref, bhh_ref,
              h0_ref, x0_ref, deg_ref, xws0_ref, xws1_ref, xws2_ref,
              xws3_ref, dis_ref):
    xt = xtr_ref[...] * tv_ref[...]
    gi = lax.dot_general(xt, wih_ref[...], (((1,), (1,)), ((), ())),
                         preferred_element_type=jnp.float32) + bih_ref[...]
    h0 = h0_ref[...]
    gh = lax.dot_general(h0, whh_ref[...], (((1,), (1,)), ((), ())),
                         preferred_element_type=jnp.float32) + bhh_ref[...]
    r = jax.nn.sigmoid(gi[:, :H] + gh[:, :H])
    z = jax.nn.sigmoid(gi[:, H:2 * H] + gh[:, H:2 * H])
    nn_ = jnp.tanh(gi[:, 2 * H:] + r * gh[:, 2 * H:])
    W = (1.0 - z) * nn_ + z * h0
    xw = lax.dot_general(x0_ref[...], W, (((1,), (0,)), ((), ())),
                         preferred_element_type=jnp.float32)
    deg = deg_ref[0] + deg_ref[1]
    degsum = jnp.sum(deg, axis=1, keepdims=True)[:N] * (1.0 / 128.0) + 1.0
    dis = lax.rsqrt(degsum)
    xws = xw * dis
    xws0_ref[...] = xws[:, 0 * WQ:1 * WQ]
    xws1_ref[...] = xws[:, 1 * WQ:2 * WQ]
    xws2_ref[...] = xws[:, 2 * WQ:3 * WQ]
    xws3_ref[...] = xws[:, 3 * WQ:4 * WQ]
    dis_ref[...] = dis


def _tc2(xperm, topv2, wih, whh, bih2, bhh2, h0, x0, deg128):
    return pl.pallas_call(
        _tc2_body,
        out_shape=[jax.ShapeDtypeStruct((N, WQ), jnp.float32)] * NQ
        + [jax.ShapeDtypeStruct((N, 1), jnp.float32)],
    )(xperm, topv2, wih, whh, bih2, bhh2, h0, x0, deg128)


def _tc3_body(y0_ref, y1_ref, y2_ref, y3_ref, xws0_ref, xws1_ref, xws2_ref,
              xws3_ref, dis_ref, fw_ref, fb_ref, h_ref, o_ref):
    parts = []
    for y_ref, xws_ref in ((y0_ref, xws0_ref), (y1_ref, xws1_ref),
                           (y2_ref, xws2_ref), (y3_ref, xws3_ref)):
        parts.append(y_ref[0, :N, :] + y_ref[1, :N, :] + xws_ref[...])
    h = dis_ref[...] * jnp.concatenate(parts, axis=1)
    h_ref[...] = h
    o_ref[...] = jnp.sum(jnp.maximum(h, 0.0) * fw_ref[...],
                         axis=1, keepdims=True) + fb_ref[0, 0]


def _tc3(ys, xwss, dis, fc_W, fc_b2):
    return pl.pallas_call(
        _tc3_body,
        out_shape=[jax.ShapeDtypeStruct((N, H), jnp.float32),
                   jax.ShapeDtypeStruct((N, 1), jnp.float32)],
    )(*ys, *xwss, dis, fc_W, fc_b2)


def _sc1_body(dst_hbm, perm_hbm, x0_hbm, zerosQ_hbm, ones128_hbm,
              deg_out, xperm_out, didx_v, ones_v, pidx_v, prow_v, deg_sh, sem):
    c = lax.axis_index("c")
    s = lax.axis_index("s")
    wid = s * 2 + c
    pltpu.sync_copy(zerosQ_hbm.at[pl.ds(s * RPS, RPS)],
                    deg_sh.at[pl.ds(s * RPS, RPS)])
    pltpu.sync_copy(ones128_hbm, ones_v)
    plsc.subcore_barrier()
    base = wid * EPW

    def body(i, carry):
        pltpu.sync_copy(dst_hbm.at[pl.ds(base + i * EB, EB)], didx_v)
        pltpu.sync_copy(ones_v, deg_sh.at[didx_v], add=True)
        return carry

    lax.fori_loop(0, NB, body, 0)
    plsc.subcore_barrier()
    pltpu.sync_copy(deg_sh.at[pl.ds(s * RPS, RPS)],
                    deg_out.at[c, pl.ds(s * RPS, RPS)])
    pltpu.sync_copy(perm_hbm.at[pl.ds(wid * 16, 16)], pidx_v)
    pltpu.async_copy(x0_hbm.at[pidx_v], prow_v, sem).wait()
    pltpu.sync_copy(prow_v, xperm_out.at[pl.ds(wid * 16, 16)])


_sc1 = pl.kernel(
    _sc1_body,
    out_type=[jax.ShapeDtypeStruct((2, NPAD, 128), jnp.float32),
              jax.ShapeDtypeStruct((H, F), jnp.float32)],
    mesh=_MESH,
    scratch_types=[pltpu.VMEM((EB,), jnp.int32),
                   pltpu.VMEM((EB, 128), jnp.float32),
                   pltpu.VMEM((16,), jnp.int32),
                   pltpu.VMEM((16, F), jnp.float32),
                   pltpu.VMEM_SHARED((NPAD, 128), jnp.float32),
                   pltpu.SemaphoreType.DMA],
)


def _sc2_body(src_hbm, dst_hbm, xws0_hbm, xws1_hbm, xws2_hbm, xws3_hbm,
              zerosQ_hbm, y0_out, y1_out, y2_out, y3_out,
              sidx_v, didx_v, rows_v, y_sh, sem):
    c = lax.axis_index("c")
    s = lax.axis_index("s")
    wid = s * 2 + c
    base = wid * EPW
    for xws_hbm, y_out in ((xws0_hbm, y0_out), (xws1_hbm, y1_out),
                           (xws2_hbm, y2_out), (xws3_hbm, y3_out)):
        pltpu.sync_copy(zerosQ_hbm.at[pl.ds(s * RPS, RPS)],
                        y_sh.at[pl.ds(s * RPS, RPS)])
        plsc.subcore_barrier()

        def body(i, carry):
            pltpu.sync_copy(src_hbm.at[pl.ds(base + i * EB, EB)], sidx_v)
            pltpu.sync_copy(dst_hbm.at[pl.ds(base + i * EB, EB)], didx_v)
            pltpu.async_copy(xws_hbm.at[sidx_v], rows_v, sem).wait()
            pltpu.sync_copy(rows_v, y_sh.at[didx_v], add=True)
            return carry

        lax.fori_loop(0, NB, body, 0)
        plsc.subcore_barrier()
        pltpu.sync_copy(y_sh.at[pl.ds(s * RPS, RPS)],
                        y_out.at[c, pl.ds(s * RPS, RPS)])


_sc2 = pl.kernel(
    _sc2_body,
    out_type=[jax.ShapeDtypeStruct((2, NPAD, WQ), jnp.float32)] * NQ,
    mesh=_MESH,
    scratch_types=[pltpu.VMEM((EB,), jnp.int32),
                   pltpu.VMEM((EB,), jnp.int32),
                   pltpu.VMEM((EB, WQ), jnp.float32),
                   pltpu.VMEM_SHARED((NPAD, WQ), jnp.float32),
                   pltpu.SemaphoreType.DMA],
)


@jax.jit
def kernel(x, edge_index, fc0_W, fc0_b, pool_p, gru_Wih, gru_Whh,
           gru_bih, gru_bhh, init_W, fc_W, fc_b):
    pad = EPAD - E
    src = jnp.concatenate([edge_index[0],
                           jnp.zeros((pad,), edge_index.dtype)])
    dst = jnp.concatenate([edge_index[1],
                           jnp.full((pad,), NPAD - 1, edge_index.dtype)])
    ones128 = jnp.ones((EB, 128), jnp.float32)
    zerosQ = jnp.zeros((NPAD, WQ), jnp.float32)

    x0s = jax.nn.relu(x @ fc0_W.T + fc0_b)
    score = jnp.tanh((x0s @ pool_p) / jnp.linalg.norm(pool_p))
    topv, perm = lax.top_k(score, H)

    x0 = _tc1(x, fc0_W, fc0_b.reshape(1, H))
    deg128, xperm = _sc1(dst, perm, x0, zerosQ, ones128)
    *xwss, dis = _tc2(xperm, topv.reshape(H, 1),
                      gru_Wih, gru_Whh, gru_bih.reshape(1, 3 * H),
                      gru_bhh.reshape(1, 3 * H), init_W[0], x0, deg128)
    ys = _sc2(src, dst, *xwss, zerosQ)
    h, out2 = _tc3(ys, xwss, dis, fc_W, fc_b.reshape(1, 1))
    return (out2[:, 0], h)


# speedup vs baseline: 5.6155x; 1.2194x over previous
import functools

import jax
import jax.numpy as jnp
from jax import lax
from jax.experimental import pallas as pl
from jax.experimental.pallas import tpu as pltpu
from jax.experimental.pallas import tpu_sc as plsc

N = 3000
F = 512
H = 512
E = 96000
NPAD = 3072
EPAD = 98304
NW = 32
EPW = EPAD // NW
EB = 128
NB = EPW // EB
RPS = NPAD // 16
NQ = 4
WQ = H // NQ

_MESH = plsc.VectorSubcoreMesh(core_axis_name="c", subcore_axis_name="s")


def _tc1_body(x_ref, w_ref, b_ref, x0_ref):
    x0 = lax.dot_general(x_ref[...], w_ref[...], (((1,), (1,)), ((), ())),
                         preferred_element_type=jnp.float32)
    x0_ref[...] = jnp.maximum(x0 + b_ref[...], 0.0)


def _tc1(x, fc0_W, fc0_b2):
    return pl.pallas_call(
        _tc1_body,
        out_shape=jax.ShapeDtypeStruct((N, F), jnp.float32),
    )(x, fc0_W, fc0_b2)


def _tc2_body(xtr_ref, tv_ref, wih_ref, whh_ref, bih_ref, bhh_ref,
              h0_ref, x0_ref, deg_ref, xws0_ref, xws1_ref, xws2_ref,
              xws3_ref, dis_ref):
    xt = xtr_ref[...] * tv_ref[...]
    gi = lax.dot_general(xt, wih_ref[...], (((1,), (1,)), ((), ())),
                         preferred_element_type=jnp.float32) + bih_ref[...]
    h0 = h0_ref[...]
    gh = lax.dot_general(h0, whh_ref[...], (((1,), (1,)), ((), ())),
                         preferred_element_type=jnp.float32) + bhh_ref[...]
    r = jax.nn.sigmoid(gi[:, :H] + gh[:, :H])
    z = jax.nn.sigmoid(gi[:, H:2 * H] + gh[:, H:2 * H])
    nn_ = jnp.tanh(gi[:, 2 * H:] + r * gh[:, 2 * H:])
    W = (1.0 - z) * nn_ + z * h0
    xw = lax.dot_general(x0_ref[...], W, (((1,), (0,)), ((), ())),
                         preferred_element_type=jnp.float32)
    deg = deg_ref[0] + deg_ref[1]
    degsum = jnp.sum(deg, axis=1, keepdims=True)[:N] * (1.0 / 128.0) + 1.0
    dis = lax.rsqrt(degsum)
    xws = xw * dis
    xws0_ref[...] = xws[:, 0 * WQ:1 * WQ]
    xws1_ref[...] = xws[:, 1 * WQ:2 * WQ]
    xws2_ref[...] = xws[:, 2 * WQ:3 * WQ]
    xws3_ref[...] = xws[:, 3 * WQ:4 * WQ]
    dis_ref[...] = dis


def _tc2(xperm, topv2, wih, whh, bih2, bhh2, h0, x0, deg128):
    return pl.pallas_call(
        _tc2_body,
        out_shape=[jax.ShapeDtypeStruct((N, WQ), jnp.float32)] * NQ
        + [jax.ShapeDtypeStruct((N, 1), jnp.float32)],
    )(xperm, topv2, wih, whh, bih2, bhh2, h0, x0, deg128)


def _tc3_body(y0_ref, y1_ref, y2_ref, y3_ref, xws0_ref, xws1_ref, xws2_ref,
              xws3_ref, dis_ref, fw_ref, fb_ref, h_ref, o_ref):
    parts = []
    for y_ref, xws_ref in ((y0_ref, xws0_ref), (y1_ref, xws1_ref),
                           (y2_ref, xws2_ref), (y3_ref, xws3_ref)):
        parts.append(y_ref[0, :N, :] + y_ref[1, :N, :] + xws_ref[...])
    h = dis_ref[...] * jnp.concatenate(parts, axis=1)
    h_ref[...] = h
    o128 = lax.dot_general(jnp.maximum(h, 0.0), fw_ref[...],
                           (((1,), (1,)), ((), ())),
                           preferred_element_type=jnp.float32)
    o_ref[...] = o128[:, :1] + fb_ref[0, 0]


def _tc3(ys, xwss, dis, fc_Wpad, fc_b2):
    return pl.pallas_call(
        _tc3_body,
        out_shape=[jax.ShapeDtypeStruct((N, H), jnp.float32),
                   jax.ShapeDtypeStruct((N, 1), jnp.float32)],
    )(*ys, *xwss, dis, fc_Wpad, fc_b2)


def _sc1_body(dst2_hbm, perm_hbm, x0_hbm, zerosQ_hbm, ones128_hbm,
              deg_out, xperm_out, didx_v, ones_v, pidx_v, prow_v, deg_sh,
              sem, psem):
    c = lax.axis_index("c")
    s = lax.axis_index("s")
    wid = s * 2 + c
    pltpu.sync_copy(zerosQ_hbm.at[pl.ds(s * RPS, RPS)],
                    deg_sh.at[pl.ds(s * RPS, RPS)])
    pltpu.sync_copy(ones128_hbm, ones_v)
    pltpu.sync_copy(dst2_hbm.at[pl.ds(wid * NB, NB)], didx_v)
    pltpu.sync_copy(perm_hbm.at[pl.ds(wid * 16, 16)], pidx_v)
    gcp = pltpu.async_copy(x0_hbm.at[pidx_v], prow_v, psem)
    plsc.subcore_barrier()

    def fire(i, carry):
        pltpu.async_copy(ones_v, deg_sh.at[didx_v.at[i]], sem, add=True)
        return carry

    lax.fori_loop(0, NB, fire, 0)

    def drain(i, carry):
        pltpu.make_async_copy(ones_v, deg_sh.at[didx_v.at[i]], sem).wait()
        return carry

    lax.fori_loop(0, NB, drain, 0)
    plsc.subcore_barrier()
    pltpu.sync_copy(deg_sh.at[pl.ds(s * RPS, RPS)],
                    deg_out.at[c, pl.ds(s * RPS, RPS)])
    gcp.wait()
    pltpu.sync_copy(prow_v, xperm_out.at[pl.ds(wid * 16, 16)])


_sc1 = pl.kernel(
    _sc1_body,
    out_type=[jax.ShapeDtypeStruct((2, NPAD, 128), jnp.float32),
              jax.ShapeDtypeStruct((H, F), jnp.float32)],
    mesh=_MESH,
    scratch_types=[pltpu.VMEM((NB, EB), jnp.int32),
                   pltpu.VMEM((EB, 128), jnp.float32),
                   pltpu.VMEM((16,), jnp.int32),
                   pltpu.VMEM((16, F), jnp.float32),
                   pltpu.VMEM_SHARED((NPAD, 128), jnp.float32),
                   pltpu.SemaphoreType.DMA,
                   pltpu.SemaphoreType.DMA],
)


def _sc2_body(src2_hbm, dst2_hbm, xws0_hbm, xws1_hbm, xws2_hbm, xws3_hbm,
              zerosQ_hbm, y0_out, y1_out, y2_out, y3_out,
              sidx_v, didx_v, rows_v, y_sh, sem0, sem1):
    c = lax.axis_index("c")
    s = lax.axis_index("s")
    wid = s * 2 + c
    pltpu.sync_copy(src2_hbm.at[pl.ds(wid * NB, NB)], sidx_v)
    pltpu.sync_copy(dst2_hbm.at[pl.ds(wid * NB, NB)], didx_v)
    for xws_hbm, y_out in ((xws0_hbm, y0_out), (xws1_hbm, y1_out),
                           (xws2_hbm, y2_out), (xws3_hbm, y3_out)):
        pltpu.sync_copy(zerosQ_hbm.at[pl.ds(s * RPS, RPS)],
                        y_sh.at[pl.ds(s * RPS, RPS)])
        plsc.subcore_barrier()
        pltpu.async_copy(xws_hbm.at[sidx_v.at[0]], rows_v.at[0], sem0)

        def body(j, carry):
            i0 = 2 * j
            pltpu.async_copy(xws_hbm.at[sidx_v.at[i0 + 1]], rows_v.at[1],
                             sem1)
            pltpu.make_async_copy(xws_hbm.at[sidx_v.at[i0]], rows_v.at[0],
                                  sem0).wait()
            pltpu.sync_copy(rows_v.at[0], y_sh.at[didx_v.at[i0]], add=True)

            @pl.when(j < NB // 2 - 1)
            def _():
                pltpu.async_copy(xws_hbm.at[sidx_v.at[i0 + 2]], rows_v.at[0],
                                 sem0)

            pltpu.make_async_copy(xws_hbm.at[sidx_v.at[i0 + 1]],
                                  rows_v.at[1], sem1).wait()
            pltpu.sync_copy(rows_v.at[1], y_sh.at[didx_v.at[i0 + 1]],
                            add=True)
            return carry

        lax.fori_loop(0, NB // 2, body, 0)
        plsc.subcore_barrier()
        pltpu.sync_copy(y_sh.at[pl.ds(s * RPS, RPS)],
                        y_out.at[c, pl.ds(s * RPS, RPS)])


_sc2 = pl.kernel(
    _sc2_body,
    out_type=[jax.ShapeDtypeStruct((2, NPAD, WQ), jnp.float32)] * NQ,
    mesh=_MESH,
    scratch_types=[pltpu.VMEM((NB, EB), jnp.int32),
                   pltpu.VMEM((NB, EB), jnp.int32),
                   pltpu.VMEM((2, EB, WQ), jnp.float32),
                   pltpu.VMEM_SHARED((NPAD, WQ), jnp.float32),
                   pltpu.SemaphoreType.DMA,
                   pltpu.SemaphoreType.DMA],
)


@jax.jit
def kernel(x, edge_index, fc0_W, fc0_b, pool_p, gru_Wih, gru_Whh,
           gru_bih, gru_bhh, init_W, fc_W, fc_b):
    pad = EPAD - E
    src2 = jnp.concatenate([edge_index[0],
                            jnp.zeros((pad,), edge_index.dtype)]
                           ).reshape(NW * NB, EB)
    dst2 = jnp.concatenate([edge_index[1],
                            jnp.full((pad,), NPAD - 1, edge_index.dtype)]
                           ).reshape(NW * NB, EB)
    ones128 = jnp.ones((EB, 128), jnp.float32)
    zerosQ = jnp.zeros((NPAD, WQ), jnp.float32)

    x0s = jax.nn.relu(x @ fc0_W.T + fc0_b)
    score = jnp.tanh((x0s @ pool_p) / jnp.linalg.norm(pool_p))
    topv, perm = lax.top_k(score, H)

    x0 = _tc1(x, fc0_W, fc0_b.reshape(1, H))
    deg128, xperm = _sc1(dst2, perm, x0, zerosQ, ones128)
    *xwss, dis = _tc2(xperm, topv.reshape(H, 1),
                      gru_Wih, gru_Whh, gru_bih.reshape(1, 3 * H),
                      gru_bhh.reshape(1, 3 * H), init_W[0], x0, deg128)
    ys = _sc2(src2, dst2, *xwss, zerosQ)
    fc_Wpad = jnp.concatenate([fc_W, jnp.zeros((127, H), jnp.float32)], axis=0)
    h, out2 = _tc3(ys, xwss, dis, fc_Wpad, fc_b.reshape(1, 1))
    return (out2[:, 0], h)


# speedup vs baseline: 5.6174x; 1.0003x over previous
import jax
import jax.numpy as jnp
from jax import lax
from jax.experimental import pallas as pl
from jax.experimental.pallas import tpu as pltpu
from jax.experimental.pallas import tpu_sc as plsc

N = 3000
F = 512
H = 512
E = 96000
NPAD = 3072
EPAD = 98304
NW = 32
EPW = EPAD // NW
EB = 128
NB = EPW // EB
RPS = NPAD // 16
NQ = 4
WQ = H // NQ

_MESH = plsc.VectorSubcoreMesh(core_axis_name="c", subcore_axis_name="s")


def _tc1_body(x_ref, w_ref, b_ref, x0_ref):
    x0 = lax.dot_general(x_ref[...], w_ref[...], (((1,), (1,)), ((), ())),
                         preferred_element_type=jnp.float32)
    x0_ref[...] = jnp.maximum(x0 + b_ref[...], 0.0)


def _tc1(x, fc0_W, fc0_b2):
    return pl.pallas_call(
        _tc1_body,
        out_shape=jax.ShapeDtypeStruct((N, F), jnp.float32),
    )(x, fc0_W, fc0_b2)


def _tc2_body(xtr_ref, tv_ref, wih_ref, whh_ref, bih_ref, bhh_ref,
              h0_ref, x0_ref, deg_ref, xws0_ref, xws1_ref, xws2_ref,
              xws3_ref, dis_ref):
    xt = xtr_ref[...] * tv_ref[...]
    gi = lax.dot_general(xt, wih_ref[...], (((1,), (1,)), ((), ())),
                         preferred_element_type=jnp.float32) + bih_ref[...]
    h0 = h0_ref[...]
    gh = lax.dot_general(h0, whh_ref[...], (((1,), (1,)), ((), ())),
                         preferred_element_type=jnp.float32) + bhh_ref[...]
    r = jax.nn.sigmoid(gi[:, :H] + gh[:, :H])
    z = jax.nn.sigmoid(gi[:, H:2 * H] + gh[:, H:2 * H])
    nn_ = jnp.tanh(gi[:, 2 * H:] + r * gh[:, 2 * H:])
    W = (1.0 - z) * nn_ + z * h0
    xw = lax.dot_general(x0_ref[...], W, (((1,), (0,)), ((), ())),
                         preferred_element_type=jnp.float32)
    deg = deg_ref[0] + deg_ref[1]
    degsum = jnp.sum(deg, axis=1, keepdims=True)[:N] * (1.0 / 128.0) + 1.0
    dis = lax.rsqrt(degsum)
    xws = xw * dis
    xws0_ref[...] = xws[:, 0 * WQ:1 * WQ]
    xws1_ref[...] = xws[:, 1 * WQ:2 * WQ]
    xws2_ref[...] = xws[:, 2 * WQ:3 * WQ]
    xws3_ref[...] = xws[:, 3 * WQ:4 * WQ]
    dis_ref[...] = dis


def _tc2(xperm, topv2, wih, whh, bih2, bhh2, h0, x0, deg128):
    return pl.pallas_call(
        _tc2_body,
        out_shape=[jax.ShapeDtypeStruct((N, WQ), jnp.float32)] * NQ
        + [jax.ShapeDtypeStruct((N, 1), jnp.float32)],
    )(xperm, topv2, wih, whh, bih2, bhh2, h0, x0, deg128)


def _tc3_body(y0_ref, y1_ref, y2_ref, y3_ref, xws0_ref, xws1_ref, xws2_ref,
              xws3_ref, dis_ref, fw_ref, fb_ref, h_ref, o_ref):
    parts = []
    for y_ref, xws_ref in ((y0_ref, xws0_ref), (y1_ref, xws1_ref),
                           (y2_ref, xws2_ref), (y3_ref, xws3_ref)):
        parts.append(y_ref[0, :N, :] + y_ref[1, :N, :] + xws_ref[...])
    h = dis_ref[...] * jnp.concatenate(parts, axis=1)
    h_ref[...] = h
    o128 = lax.dot_general(jnp.maximum(h, 0.0), fw_ref[...],
                           (((1,), (1,)), ((), ())),
                           preferred_element_type=jnp.float32)
    o_ref[...] = o128[:, :1] + fb_ref[0, 0]


def _tc3(ys, xwss, dis, fc_Wpad, fc_b2):
    return pl.pallas_call(
        _tc3_body,
        out_shape=[jax.ShapeDtypeStruct((N, H), jnp.float32),
                   jax.ShapeDtypeStruct((N, 1), jnp.float32)],
    )(*ys, *xwss, dis, fc_Wpad, fc_b2)


def _sc1_body(dst2_hbm, perm_hbm, x0_hbm, zerosQ_hbm, ones128_hbm,
              deg_out, xperm_out, didx_v, ones_v, pidx_v, prow_v, deg_sh,
              sem, psem):
    c = lax.axis_index("c")
    s = lax.axis_index("s")
    wid = s * 2 + c
    pltpu.sync_copy(zerosQ_hbm.at[pl.ds(s * RPS, RPS)],
                    deg_sh.at[pl.ds(s * RPS, RPS)])
    pltpu.sync_copy(ones128_hbm, ones_v)
    pltpu.sync_copy(dst2_hbm.at[pl.ds(wid * NB, NB)], didx_v)
    pltpu.sync_copy(perm_hbm.at[pl.ds(wid * 16, 16)], pidx_v)
    gcp = pltpu.async_copy(x0_hbm.at[pidx_v], prow_v, psem)
    plsc.subcore_barrier()

    def fire(i, carry):
        pltpu.async_copy(ones_v, deg_sh.at[didx_v.at[i]], sem, add=True)
        return carry

    lax.fori_loop(0, NB, fire, 0)

    def drain(i, carry):
        pltpu.make_async_copy(ones_v, deg_sh.at[didx_v.at[i]], sem).wait()
        return carry

    lax.fori_loop(0, NB, drain, 0)
    plsc.subcore_barrier()
    pltpu.sync_copy(deg_sh.at[pl.ds(s * RPS, RPS)],
                    deg_out.at[c, pl.ds(s * RPS, RPS)])
    gcp.wait()
    pltpu.sync_copy(prow_v, xperm_out.at[pl.ds(wid * 16, 16)])


_sc1 = pl.kernel(
    _sc1_body,
    out_type=[jax.ShapeDtypeStruct((2, NPAD, 128), jnp.float32),
              jax.ShapeDtypeStruct((H, F), jnp.float32)],
    mesh=_MESH,
    scratch_types=[pltpu.VMEM((NB, EB), jnp.int32),
                   pltpu.VMEM((EB, 128), jnp.float32),
                   pltpu.VMEM((16,), jnp.int32),
                   pltpu.VMEM((16, F), jnp.float32),
                   pltpu.VMEM_SHARED((NPAD, 128), jnp.float32),
                   pltpu.SemaphoreType.DMA,
                   pltpu.SemaphoreType.DMA],
)


def _sc2_body(src2_hbm, dst2_hbm, xws0_hbm, xws1_hbm, xws2_hbm, xws3_hbm,
              zerosQ_hbm, y0_out, y1_out, y2_out, y3_out,
              sidx_v, didx_v, rows_v, y_sh, sem0, sem1):
    c = lax.axis_index("c")
    s = lax.axis_index("s")
    wid = s * 2 + c
    pltpu.sync_copy(src2_hbm.at[pl.ds(wid * NB, NB)], sidx_v)
    pltpu.sync_copy(dst2_hbm.at[pl.ds(wid * NB, NB)], didx_v)
    for xws_hbm, y_out in ((xws0_hbm, y0_out), (xws1_hbm, y1_out),
                           (xws2_hbm, y2_out), (xws3_hbm, y3_out)):
        pltpu.sync_copy(zerosQ_hbm.at[pl.ds(s * RPS, RPS)],
                        y_sh.at[pl.ds(s * RPS, RPS)])
        plsc.subcore_barrier()
        pltpu.async_copy(xws_hbm.at[sidx_v.at[0]], rows_v.at[0], sem0)

        def body(j, carry):
            i0 = 2 * j
            pltpu.async_copy(xws_hbm.at[sidx_v.at[i0 + 1]], rows_v.at[1],
                             sem1)
            pltpu.make_async_copy(xws_hbm.at[sidx_v.at[i0]], rows_v.at[0],
                                  sem0).wait()
            pltpu.sync_copy(rows_v.at[0], y_sh.at[didx_v.at[i0]], add=True)

            @pl.when(j < NB // 2 - 1)
            def _():
                pltpu.async_copy(xws_hbm.at[sidx_v.at[i0 + 2]], rows_v.at[0],
                                 sem0)

            pltpu.make_async_copy(xws_hbm.at[sidx_v.at[i0 + 1]],
                                  rows_v.at[1], sem1).wait()
            pltpu.sync_copy(rows_v.at[1], y_sh.at[didx_v.at[i0 + 1]],
                            add=True)
            return carry

        lax.fori_loop(0, NB // 2, body, 0)
        plsc.subcore_barrier()
        pltpu.sync_copy(y_sh.at[pl.ds(s * RPS, RPS)],
                        y_out.at[c, pl.ds(s * RPS, RPS)])


_sc2 = pl.kernel(
    _sc2_body,
    out_type=[jax.ShapeDtypeStruct((2, NPAD, WQ), jnp.float32)] * NQ,
    mesh=_MESH,
    scratch_types=[pltpu.VMEM((NB, EB), jnp.int32),
                   pltpu.VMEM((NB, EB), jnp.int32),
                   pltpu.VMEM((2, EB, WQ), jnp.float32),
                   pltpu.VMEM_SHARED((NPAD, WQ), jnp.float32),
                   pltpu.SemaphoreType.DMA,
                   pltpu.SemaphoreType.DMA],
)


@jax.jit
def kernel(x, edge_index, fc0_W, fc0_b, pool_p, gru_Wih, gru_Whh,
           gru_bih, gru_bhh, init_W, fc_W, fc_b):
    pad = EPAD - E
    src2 = jnp.concatenate([edge_index[0],
                            jnp.zeros((pad,), edge_index.dtype)]
                           ).reshape(NW * NB, EB)
    dst2 = jnp.concatenate([edge_index[1],
                            jnp.full((pad,), NPAD - 1, edge_index.dtype)]
                           ).reshape(NW * NB, EB)
    ones128 = jnp.ones((EB, 128), jnp.float32)
    zerosQ = jnp.zeros((NPAD, WQ), jnp.float32)

    x0s = jax.nn.relu(x @ fc0_W.T + fc0_b)
    score = jnp.tanh((x0s @ pool_p) / jnp.linalg.norm(pool_p))
    topv, perm = lax.top_k(score, H)

    x0 = _tc1(x, fc0_W, fc0_b.reshape(1, H))
    deg128, xperm = _sc1(dst2, perm, x0, zerosQ, ones128)
    *xwss, dis = _tc2(xperm, topv.reshape(H, 1),
                      gru_Wih, gru_Whh, gru_bih.reshape(1, 3 * H),
                      gru_bhh.reshape(1, 3 * H), init_W[0], x0, deg128)
    ys = _sc2(src2, dst2, *xwss, zerosQ)
    fc_Wpad = jnp.concatenate([fc_W, jnp.zeros((127, H), jnp.float32)], axis=0)
    h, out2 = _tc3(ys, xwss, dis, fc_Wpad, fc_b.reshape(1, 1))
    return (out2[:, 0], h)
